# baseline (speedup 1.0000x reference)
import functools

import jax
import jax.numpy as jnp
from jax import lax
from jax.experimental import pallas as pl
from jax.experimental.pallas import tpu as pltpu
from jax.experimental.pallas import tpu_sc as plsc

N = 10000
E = 320000
F = 128
HF = 64
EPS = 1e-7
BN_EPS = 1e-5

NC = 2
NS = 16
LANES = 16
KS = HF // LANES

B = 64
EPT = 20480
EPAD = EPT * NS
NCHUNK = EPT // B
SEC = 16
NSEC = NCHUNK // SEC
NP = 10240
NPT = NP // NS
FB = 64
NFIN = NPT // FB


def _agg_body(h2, sd_hbm, out,
              acc,
              ssec0, ssec1, dsec0, dsec1, rows0, rows1, em,
              accb, outb, ssem0, ssem1, gsem0, gsem1):
    c = lax.axis_index("c")
    s = lax.axis_index("s")
    sbufs = ((ssec0, dsec0, ssem0), (ssec1, dsec1, ssem1))
    gbufs = ((rows0, gsem0), (rows1, gsem1))

    @pl.loop(0, FB)
    def _zero(i):
        for k in range(2 * KS):
            accb[i, pl.ds(k * LANES, LANES)] = jnp.zeros((LANES,), jnp.float32)

    @pl.loop(0, NFIN)
    def _zcopy(p):
        pltpu.sync_copy(accb, acc.at[pl.ds(s * NPT + p * FB, FB)])

    plsc.subcore_barrier()

    pltpu.async_copy(sd_hbm.at[s, 0, 0], ssec0, ssem0)
    pltpu.async_copy(sd_hbm.at[s, 0, 1], dsec0, ssem0)

    @pl.loop(0, NSEC, step=2)
    def _sec(t):
        for sb in range(2):
            sec = t + sb
            ssec, dsec, ssem = sbufs[sb]
            nssec, ndsec, nssem = sbufs[1 - sb]

            pltpu.make_async_copy(sd_hbm.at[s, sec, 0], ssec, ssem).wait()
            pltpu.make_async_copy(sd_hbm.at[s, sec, 1], dsec, ssem).wait()

            @pl.when(sec + 1 < NSEC)
            def _next_sec():
                pltpu.async_copy(sd_hbm.at[s, sec + 1, 0], nssec, nssem)
                pltpu.async_copy(sd_hbm.at[s, sec + 1, 1], ndsec, nssem)

            @pl.loop(0, SEC)
            def _gidx(g):
                for k in range(B // LANES):
                    sv = ssec[g, pl.ds(k * LANES, LANES)]
                    ssec[g, pl.ds(k * LANES, LANES)] = sv * 2 + c

            pltpu.async_copy(h2.at[ssec.at[0]], rows0, gsem0)
            pltpu.async_copy(h2.at[ssec.at[1]], rows1, gsem1)

            @pl.loop(0, SEC, step=2)
            def _chunk(q):
                for rb in range(2):
                    ch = q + rb
                    rows, gsem = gbufs[rb]

                    pltpu.make_async_copy(h2.at[ssec.at[ch]], rows, gsem).wait()

                    @pl.loop(0, B)
                    def _edge(i):
                        for k in range(KS):
                            v = rows[i, pl.ds(k * LANES, LANES)]
                            m = jnp.maximum(v, 0.0) + EPS
                            e = jnp.exp(m)
                            em[i, pl.ds(k * LANES, LANES)] = e
                            em[i, pl.ds(HF + k * LANES, LANES)] = m * e

                    @pl.when(ch + 2 < SEC)
                    def _prefetch():
                        pltpu.async_copy(h2.at[ssec.at[ch + 2]], rows, gsem)

                    pltpu.sync_copy(em, acc.at[dsec.at[ch]], add=True)

    plsc.subcore_barrier()

    @pl.loop(0, NFIN)
    def _fin(p):
        nb = s * NPT + p * FB
        pltpu.sync_copy(acc.at[pl.ds(nb, FB)], accb)

        @pl.loop(0, FB)
        def _node(i):
            for k in range(KS):
                d = accb[i, pl.ds(k * LANES, LANES)]
                u = accb[i, pl.ds(HF + k * LANES, LANES)]
                outb[i, pl.ds(k * LANES, LANES)] = u / (d + 1e-16)

        pltpu.sync_copy(outb, out.at[c, pl.ds(nb, FB)])


_sc_aggregate = pl.kernel(
    _agg_body,
    out_type=jax.ShapeDtypeStruct((NC, NP, HF), jnp.float32),
    mesh=plsc.VectorSubcoreMesh(core_axis_name="c", subcore_axis_name="s"),
    compiler_params=pltpu.CompilerParams(use_tc_tiling_on_sc=False),
    scratch_types=[
        pltpu.VMEM_SHARED((NP, 2 * HF), jnp.float32),
        pltpu.VMEM((SEC, B), jnp.int32),
        pltpu.VMEM((SEC, B), jnp.int32),
        pltpu.VMEM((SEC, B), jnp.int32),
        pltpu.VMEM((SEC, B), jnp.int32),
        pltpu.VMEM((B, HF), jnp.float32),
        pltpu.VMEM((B, HF), jnp.float32),
        pltpu.VMEM((B, 2 * HF), jnp.float32),
        pltpu.VMEM((FB, 2 * HF), jnp.float32),
        pltpu.VMEM((FB, HF), jnp.float32),
        pltpu.SemaphoreType.DMA,
        pltpu.SemaphoreType.DMA,
        pltpu.SemaphoreType.DMA,
        pltpu.SemaphoreType.DMA,
    ],
)


def _mlp_body(final, aggr_ref, h_ref, wa_ref, ba_ref, gs_ref, be_ref,
              wb_ref, bb_ref, o_ref):
    a = jnp.concatenate([aggr_ref[0], aggr_ref[1]], axis=-1) + h_ref[...]
    t = jnp.dot(a, wa_ref[...], preferred_element_type=jnp.float32)
    t = t + ba_ref[...]
    t = gs_ref[...] * (t * (1.0 / jnp.sqrt(1.0 + BN_EPS))) + be_ref[...]
    t = jnp.maximum(t, 0.0)
    o = jnp.dot(t, wb_ref[...], preferred_element_type=jnp.float32)
    o = o + bb_ref[...]
    if final == "relu":
        o_ref[...] = jnp.maximum(o, 0.0)
    else:
        m = jnp.max(o, axis=1, keepdims=True)
        ex = jnp.exp(o - m)
        lse = jnp.log(jnp.sum(ex, axis=1, keepdims=True)) + m
        o_ref[...] = o - lse


def _mlp(aggr, h, wa, ba, g, be, wb, bb, final):
    bn = 1000
    fmid = wa.shape[1]
    fout = wb.shape[1]
    grid = (N // bn,)
    return pl.pallas_call(
        functools.partial(_mlp_body, final),
        grid=grid,
        in_specs=[
            pl.BlockSpec((NC, bn, HF), lambda i: (0, i, 0)),
            pl.BlockSpec((bn, F), lambda i: (i, 0)),
            pl.BlockSpec((F, fmid), lambda i: (0, 0)),
            pl.BlockSpec((1, fmid), lambda i: (0, 0)),
            pl.BlockSpec((1, fmid), lambda i: (0, 0)),
            pl.BlockSpec((1, fmid), lambda i: (0, 0)),
            pl.BlockSpec((fmid, fout), lambda i: (0, 0)),
            pl.BlockSpec((1, fout), lambda i: (0, 0)),
        ],
        out_specs=pl.BlockSpec((bn, fout), lambda i: (i, 0)),
        out_shape=jax.ShapeDtypeStruct((N, fout), jnp.float32),
    )(aggr, h, wa, ba.reshape(1, -1), g.reshape(1, -1), be.reshape(1, -1),
      wb, bb.reshape(1, -1))


def kernel(x, edge_index, W1a, b1a, g1, be1, W1b, b1b,
           W2a, b2a, g2, be2, W2b, b2b):
    srcp = jnp.pad(edge_index[0], (0, EPAD - E))
    dstp = jnp.pad(edge_index[1], (0, EPAD - E), constant_values=N)
    sd = (jnp.stack([srcp, dstp])
          .reshape(2, NS, NSEC, SEC, B)
          .transpose(1, 2, 0, 3, 4))
    aggr1 = _sc_aggregate(x.reshape(2 * N, HF), sd)[:, :N]
    h1 = _mlp(aggr1, x, W1a, b1a, g1, be1, W1b, b1b, final="relu")
    aggr2 = _sc_aggregate(h1.reshape(2 * N, HF), sd)[:, :N]
    return _mlp(aggr2, h1, W2a, b2a, g2, be2, W2b, b2b, final="logsoftmax")


# speedup vs baseline: 1.0031x; 1.0031x over previous
import functools

import jax
import jax.numpy as jnp
from jax import lax
from jax.experimental import pallas as pl
from jax.experimental.pallas import tpu as pltpu
from jax.experimental.pallas import tpu_sc as plsc

N = 10000
E = 320000
F = 128
HF = 64
EPS = 1e-7
BN_EPS = 1e-5

NC = 2
NS = 16
LANES = 16
KS = HF // LANES

B = 64
EPT = 20480
EPAD = EPT * NS
NCHUNK = EPT // B
SEC = 16
NSEC = NCHUNK // SEC
NP = 10240
NPT = NP // NS
FB = 64
NFIN = NPT // FB


def _agg_body(h2, sd_hbm, out,
              acc,
              ssec0, ssec1, dsec0, dsec1, rows0, rows1, em,
              accb, outb, ssem0, ssem1, gsem0, gsem1):
    c = lax.axis_index("c")
    s = lax.axis_index("s")
    sbufs = ((ssec0, dsec0, ssem0), (ssec1, dsec1, ssem1))
    gbufs = ((rows0, gsem0), (rows1, gsem1))

    @pl.loop(0, FB)
    def _zero(i):
        for k in range(2 * KS):
            accb[i, pl.ds(k * LANES, LANES)] = jnp.zeros((LANES,), jnp.float32)

    @pl.loop(0, NFIN)
    def _zcopy(p):
        pltpu.sync_copy(accb, acc.at[pl.ds(s * NPT + p * FB, FB)])

    plsc.subcore_barrier()

    pltpu.async_copy(sd_hbm.at[s, 0, 0], ssec0, ssem0)
    pltpu.async_copy(sd_hbm.at[s, 0, 1], dsec0, ssem0)

    @pl.loop(0, NSEC, step=2)
    def _sec(t):
        for sb in range(2):
            sec = t + sb
            ssec, dsec, ssem = sbufs[sb]
            nssec, ndsec, nssem = sbufs[1 - sb]

            pltpu.make_async_copy(sd_hbm.at[s, sec, 0], ssec, ssem).wait()
            pltpu.make_async_copy(sd_hbm.at[s, sec, 1], dsec, ssem).wait()

            @pl.when(sec + 1 < NSEC)
            def _next_sec():
                pltpu.async_copy(sd_hbm.at[s, sec + 1, 0], nssec, nssem)
                pltpu.async_copy(sd_hbm.at[s, sec + 1, 1], ndsec, nssem)

            @pl.loop(0, SEC)
            def _gidx(g):
                for k in range(B // LANES):
                    sv = ssec[g, pl.ds(k * LANES, LANES)]
                    ssec[g, pl.ds(k * LANES, LANES)] = sv * 2 + c

            pltpu.async_copy(h2.at[ssec.at[0]], rows0, gsem0)
            pltpu.async_copy(h2.at[ssec.at[1]], rows1, gsem1)

            @pl.loop(0, SEC, step=2)
            def _chunk(q):
                for rb in range(2):
                    ch = q + rb
                    rows, gsem = gbufs[rb]

                    pltpu.make_async_copy(h2.at[ssec.at[ch]], rows, gsem).wait()

                    @pl.loop(0, B, unroll=8)
                    def _edge(i):
                        for k in range(KS):
                            v = rows[i, pl.ds(k * LANES, LANES)]
                            m = jnp.maximum(v, 0.0) + EPS
                            e = jnp.exp(m)
                            em[i, pl.ds(k * LANES, LANES)] = e
                            em[i, pl.ds(HF + k * LANES, LANES)] = m * e

                    @pl.when(ch + 2 < SEC)
                    def _prefetch():
                        pltpu.async_copy(h2.at[ssec.at[ch + 2]], rows, gsem)

                    pltpu.sync_copy(em, acc.at[dsec.at[ch]], add=True)

    plsc.subcore_barrier()

    @pl.loop(0, NFIN)
    def _fin(p):
        nb = s * NPT + p * FB
        pltpu.sync_copy(acc.at[pl.ds(nb, FB)], accb)

        @pl.loop(0, FB)
        def _node(i):
            for k in range(KS):
                d = accb[i, pl.ds(k * LANES, LANES)]
                u = accb[i, pl.ds(HF + k * LANES, LANES)]
                outb[i, pl.ds(k * LANES, LANES)] = u / (d + 1e-16)

        pltpu.sync_copy(outb, out.at[c, pl.ds(nb, FB)])


_sc_aggregate = pl.kernel(
    _agg_body,
    out_type=jax.ShapeDtypeStruct((NC, NP, HF), jnp.float32),
    mesh=plsc.VectorSubcoreMesh(core_axis_name="c", subcore_axis_name="s"),
    compiler_params=pltpu.CompilerParams(use_tc_tiling_on_sc=False),
    scratch_types=[
        pltpu.VMEM_SHARED((NP, 2 * HF), jnp.float32),
        pltpu.VMEM((SEC, B), jnp.int32),
        pltpu.VMEM((SEC, B), jnp.int32),
        pltpu.VMEM((SEC, B), jnp.int32),
        pltpu.VMEM((SEC, B), jnp.int32),
        pltpu.VMEM((B, HF), jnp.float32),
        pltpu.VMEM((B, HF), jnp.float32),
        pltpu.VMEM((B, 2 * HF), jnp.float32),
        pltpu.VMEM((FB, 2 * HF), jnp.float32),
        pltpu.VMEM((FB, HF), jnp.float32),
        pltpu.SemaphoreType.DMA,
        pltpu.SemaphoreType.DMA,
        pltpu.SemaphoreType.DMA,
        pltpu.SemaphoreType.DMA,
    ],
)


def _mlp_body(final, aggr_ref, h_ref, wa_ref, ba_ref, gs_ref, be_ref,
              wb_ref, bb_ref, o_ref):
    a = jnp.concatenate([aggr_ref[0], aggr_ref[1]], axis=-1) + h_ref[...]
    t = jnp.dot(a, wa_ref[...], preferred_element_type=jnp.float32)
    t = t + ba_ref[...]
    t = gs_ref[...] * (t * (1.0 / jnp.sqrt(1.0 + BN_EPS))) + be_ref[...]
    t = jnp.maximum(t, 0.0)
    o = jnp.dot(t, wb_ref[...], preferred_element_type=jnp.float32)
    o = o + bb_ref[...]
    if final == "relu":
        o_ref[...] = jnp.maximum(o, 0.0)
    else:
        m = jnp.max(o, axis=1, keepdims=True)
        ex = jnp.exp(o - m)
        lse = jnp.log(jnp.sum(ex, axis=1, keepdims=True)) + m
        o_ref[...] = o - lse


def _mlp(aggr, h, wa, ba, g, be, wb, bb, final):
    bn = 1000
    fmid = wa.shape[1]
    fout = wb.shape[1]
    grid = (N // bn,)
    return pl.pallas_call(
        functools.partial(_mlp_body, final),
        grid=grid,
        in_specs=[
            pl.BlockSpec((NC, bn, HF), lambda i: (0, i, 0)),
            pl.BlockSpec((bn, F), lambda i: (i, 0)),
            pl.BlockSpec((F, fmid), lambda i: (0, 0)),
            pl.BlockSpec((1, fmid), lambda i: (0, 0)),
            pl.BlockSpec((1, fmid), lambda i: (0, 0)),
            pl.BlockSpec((1, fmid), lambda i: (0, 0)),
            pl.BlockSpec((fmid, fout), lambda i: (0, 0)),
            pl.BlockSpec((1, fout), lambda i: (0, 0)),
        ],
        out_specs=pl.BlockSpec((bn, fout), lambda i: (i, 0)),
        out_shape=jax.ShapeDtypeStruct((N, fout), jnp.float32),
    )(aggr, h, wa, ba.reshape(1, -1), g.reshape(1, -1), be.reshape(1, -1),
      wb, bb.reshape(1, -1))


def kernel(x, edge_index, W1a, b1a, g1, be1, W1b, b1b,
           W2a, b2a, g2, be2, W2b, b2b):
    srcp = jnp.pad(edge_index[0], (0, EPAD - E))
    dstp = jnp.pad(edge_index[1], (0, EPAD - E), constant_values=N)
    sd = (jnp.stack([srcp, dstp])
          .reshape(2, NS, NSEC, SEC, B)
          .transpose(1, 2, 0, 3, 4))
    aggr1 = _sc_aggregate(x.reshape(2 * N, HF), sd)[:, :N]
    h1 = _mlp(aggr1, x, W1a, b1a, g1, be1, W1b, b1b, final="relu")
    aggr2 = _sc_aggregate(h1.reshape(2 * N, HF), sd)[:, :N]
    return _mlp(aggr2, h1, W2a, b2a, g2, be2, W2b, b2b, final="logsoftmax")


# speedup vs baseline: 2.3567x; 2.3495x over previous
import functools

import jax
import jax.numpy as jnp
from jax import lax
from jax.experimental import pallas as pl
from jax.experimental.pallas import tpu as pltpu
from jax.experimental.pallas import tpu_sc as plsc

N = 10000
E = 320000
F = 128
HF = 64
EPS = 1e-7
BN_EPS = 1e-5

NC = 2
NS = 16
LANES = 16
KS = HF // LANES

B = 64
EPT = 20480
EPAD = EPT * NS
NCHUNK = EPT // B
SEC = 16
NSEC = NCHUNK // SEC
NP = 10240
NPT = NP // NS
FB = 64
NFIN = NPT // FB


def _agg_body(h2, sd_hbm, out,
              acc,
              ssec0, ssec1, dsec0, dsec1, rows0, rows1, em,
              accb, outb, ssem0, ssem1, gsem0, gsem1):
    c = lax.axis_index("c")
    s = lax.axis_index("s")
    sbufs = ((ssec0, dsec0, ssem0), (ssec1, dsec1, ssem1))
    gbufs = ((rows0, gsem0), (rows1, gsem1))

    @pl.loop(0, FB)
    def _zero(i):
        for k in range(2 * KS):
            accb[i, pl.ds(k * LANES, LANES)] = jnp.zeros((LANES,), jnp.float32)

    @pl.loop(0, NFIN)
    def _zcopy(p):
        pltpu.sync_copy(accb, acc.at[pl.ds(s * NPT + p * FB, FB)])

    plsc.subcore_barrier()

    pltpu.async_copy(sd_hbm.at[s, 0, 0], ssec0, ssem0)
    pltpu.async_copy(sd_hbm.at[s, 0, 1], dsec0, ssem0)

    @pl.loop(0, NSEC, step=2)
    def _sec(t):
        for sb in range(2):
            sec = t + sb
            ssec, dsec, ssem = sbufs[sb]
            nssec, ndsec, nssem = sbufs[1 - sb]

            pltpu.make_async_copy(sd_hbm.at[s, sec, 0], ssec, ssem).wait()
            pltpu.make_async_copy(sd_hbm.at[s, sec, 1], dsec, ssem).wait()

            @pl.when(sec + 1 < NSEC)
            def _next_sec():
                pltpu.async_copy(sd_hbm.at[s, sec + 1, 0], nssec, nssem)
                pltpu.async_copy(sd_hbm.at[s, sec + 1, 1], ndsec, nssem)

            @pl.loop(0, SEC)
            def _gidx(g):
                for k in range(B // LANES):
                    sv = ssec[g, pl.ds(k * LANES, LANES)]
                    ssec[g, pl.ds(k * LANES, LANES)] = sv * 2 + c

            pltpu.async_copy(h2.at[ssec.at[0]], rows0, gsem0)
            pltpu.async_copy(h2.at[ssec.at[1]], rows1, gsem1)

            @pl.loop(0, SEC, step=2)
            def _chunk(q):
                for rb in range(2):
                    ch = q + rb
                    rows, gsem = gbufs[rb]

                    pltpu.make_async_copy(h2.at[ssec.at[ch]], rows, gsem).wait()

                    @plsc.parallel_loop(0, B, unroll=8)
                    def _edge(i):
                        for k in range(KS):
                            v = rows[i, pl.ds(k * LANES, LANES)]
                            m = jnp.maximum(v, 0.0) + EPS
                            e = jnp.exp(m)
                            em[i, pl.ds(k * LANES, LANES)] = e
                            em[i, pl.ds(HF + k * LANES, LANES)] = m * e

                    @pl.when(ch + 2 < SEC)
                    def _prefetch():
                        pltpu.async_copy(h2.at[ssec.at[ch + 2]], rows, gsem)

                    pltpu.sync_copy(em, acc.at[dsec.at[ch]], add=True)

    plsc.subcore_barrier()

    @pl.loop(0, NFIN)
    def _fin(p):
        nb = s * NPT + p * FB
        pltpu.sync_copy(acc.at[pl.ds(nb, FB)], accb)

        @pl.loop(0, FB)
        def _node(i):
            for k in range(KS):
                d = accb[i, pl.ds(k * LANES, LANES)]
                u = accb[i, pl.ds(HF + k * LANES, LANES)]
                outb[i, pl.ds(k * LANES, LANES)] = u / (d + 1e-16)

        pltpu.sync_copy(outb, out.at[c, pl.ds(nb, FB)])


_sc_aggregate = pl.kernel(
    _agg_body,
    out_type=jax.ShapeDtypeStruct((NC, NP, HF), jnp.float32),
    mesh=plsc.VectorSubcoreMesh(core_axis_name="c", subcore_axis_name="s"),
    compiler_params=pltpu.CompilerParams(use_tc_tiling_on_sc=False),
    scratch_types=[
        pltpu.VMEM_SHARED((NP, 2 * HF), jnp.float32),
        pltpu.VMEM((SEC, B), jnp.int32),
        pltpu.VMEM((SEC, B), jnp.int32),
        pltpu.VMEM((SEC, B), jnp.int32),
        pltpu.VMEM((SEC, B), jnp.int32),
        pltpu.VMEM((B, HF), jnp.float32),
        pltpu.VMEM((B, HF), jnp.float32),
        pltpu.VMEM((B, 2 * HF), jnp.float32),
        pltpu.VMEM((FB, 2 * HF), jnp.float32),
        pltpu.VMEM((FB, HF), jnp.float32),
        pltpu.SemaphoreType.DMA,
        pltpu.SemaphoreType.DMA,
        pltpu.SemaphoreType.DMA,
        pltpu.SemaphoreType.DMA,
    ],
)


def _mlp_body(final, aggr_ref, h_ref, wa_ref, ba_ref, gs_ref, be_ref,
              wb_ref, bb_ref, o_ref):
    a = jnp.concatenate([aggr_ref[0], aggr_ref[1]], axis=-1) + h_ref[...]
    t = jnp.dot(a, wa_ref[...], preferred_element_type=jnp.float32)
    t = t + ba_ref[...]
    t = gs_ref[...] * (t * (1.0 / jnp.sqrt(1.0 + BN_EPS))) + be_ref[...]
    t = jnp.maximum(t, 0.0)
    o = jnp.dot(t, wb_ref[...], preferred_element_type=jnp.float32)
    o = o + bb_ref[...]
    if final == "relu":
        o_ref[...] = jnp.maximum(o, 0.0)
    else:
        m = jnp.max(o, axis=1, keepdims=True)
        ex = jnp.exp(o - m)
        lse = jnp.log(jnp.sum(ex, axis=1, keepdims=True)) + m
        o_ref[...] = o - lse


def _mlp(aggr, h, wa, ba, g, be, wb, bb, final):
    bn = 1000
    fmid = wa.shape[1]
    fout = wb.shape[1]
    grid = (N // bn,)
    return pl.pallas_call(
        functools.partial(_mlp_body, final),
        grid=grid,
        in_specs=[
            pl.BlockSpec((NC, bn, HF), lambda i: (0, i, 0)),
            pl.BlockSpec((bn, F), lambda i: (i, 0)),
            pl.BlockSpec((F, fmid), lambda i: (0, 0)),
            pl.BlockSpec((1, fmid), lambda i: (0, 0)),
            pl.BlockSpec((1, fmid), lambda i: (0, 0)),
            pl.BlockSpec((1, fmid), lambda i: (0, 0)),
            pl.BlockSpec((fmid, fout), lambda i: (0, 0)),
            pl.BlockSpec((1, fout), lambda i: (0, 0)),
        ],
        out_specs=pl.BlockSpec((bn, fout), lambda i: (i, 0)),
        out_shape=jax.ShapeDtypeStruct((N, fout), jnp.float32),
    )(aggr, h, wa, ba.reshape(1, -1), g.reshape(1, -1), be.reshape(1, -1),
      wb, bb.reshape(1, -1))


def kernel(x, edge_index, W1a, b1a, g1, be1, W1b, b1b,
           W2a, b2a, g2, be2, W2b, b2b):
    srcp = jnp.pad(edge_index[0], (0, EPAD - E))
    dstp = jnp.pad(edge_index[1], (0, EPAD - E), constant_values=N)
    sd = (jnp.stack([srcp, dstp])
          .reshape(2, NS, NSEC, SEC, B)
          .transpose(1, 2, 0, 3, 4))
    aggr1 = _sc_aggregate(x.reshape(2 * N, HF), sd)[:, :N]
    h1 = _mlp(aggr1, x, W1a, b1a, g1, be1, W1b, b1b, final="relu")
    aggr2 = _sc_aggregate(h1.reshape(2 * N, HF), sd)[:, :N]
    return _mlp(aggr2, h1, W2a, b2a, g2, be2, W2b, b2b, final="logsoftmax")


# speedup vs baseline: 2.4705x; 1.0483x over previous
import functools

import jax
import jax.numpy as jnp
from jax import lax
from jax.experimental import pallas as pl
from jax.experimental.pallas import tpu as pltpu
from jax.experimental.pallas import tpu_sc as plsc

N = 10000
E = 320000
F = 128
HF = 64
EPS = 1e-7
BN_EPS = 1e-5

NC = 2
NS = 16
LANES = 16
KS = HF // LANES

B = 64
EPT = 20480
EPAD = EPT * NS
NCHUNK = EPT // B
SEC = 16
NSEC = NCHUNK // SEC
NP = 10240
NPT = NP // NS
FB = 64
NFIN = NPT // FB


def _agg_body(h2, sd_hbm, out,
              acc,
              ssec0, ssec1, dsec0, dsec1, rows0, rows1, em,
              accb, outb, ssem0, ssem1, gsem0, gsem1):
    c = lax.axis_index("c")
    s = lax.axis_index("s")
    sbufs = ((ssec0, dsec0, ssem0), (ssec1, dsec1, ssem1))
    gbufs = ((rows0, gsem0), (rows1, gsem1))

    @plsc.parallel_loop(0, FB, unroll=8)
    def _zero(i):
        for k in range(2 * KS):
            accb[i, pl.ds(k * LANES, LANES)] = jnp.zeros((LANES,), jnp.float32)

    @pl.loop(0, NFIN)
    def _zcopy(p):
        pltpu.sync_copy(accb, acc.at[pl.ds(s * NPT + p * FB, FB)])

    plsc.subcore_barrier()

    pltpu.async_copy(sd_hbm.at[s, 0, 0], ssec0, ssem0)
    pltpu.async_copy(sd_hbm.at[s, 0, 1], dsec0, ssem0)

    @pl.loop(0, NSEC, step=2)
    def _sec(t):
        for sb in range(2):
            sec = t + sb
            ssec, dsec, ssem = sbufs[sb]
            nssec, ndsec, nssem = sbufs[1 - sb]

            pltpu.make_async_copy(sd_hbm.at[s, sec, 0], ssec, ssem).wait()
            pltpu.make_async_copy(sd_hbm.at[s, sec, 1], dsec, ssem).wait()

            @pl.when(sec + 1 < NSEC)
            def _next_sec():
                pltpu.async_copy(sd_hbm.at[s, sec + 1, 0], nssec, nssem)
                pltpu.async_copy(sd_hbm.at[s, sec + 1, 1], ndsec, nssem)

            @plsc.parallel_loop(0, SEC, unroll=4)
            def _gidx(g):
                for k in range(B // LANES):
                    sv = ssec[g, pl.ds(k * LANES, LANES)]
                    ssec[g, pl.ds(k * LANES, LANES)] = sv * 2 + c

            pltpu.async_copy(h2.at[ssec.at[0]], rows0, gsem0)
            pltpu.async_copy(h2.at[ssec.at[1]], rows1, gsem1)

            @pl.loop(0, SEC, step=2)
            def _chunk(q):
                for rb in range(2):
                    ch = q + rb
                    rows, gsem = gbufs[rb]

                    pltpu.make_async_copy(h2.at[ssec.at[ch]], rows, gsem).wait()

                    @plsc.parallel_loop(0, B, unroll=8)
                    def _edge(i):
                        for k in range(KS):
                            v = rows[i, pl.ds(k * LANES, LANES)]
                            m = jnp.maximum(v, 0.0) + EPS
                            e = jnp.exp(m)
                            em[i, pl.ds(k * LANES, LANES)] = e
                            em[i, pl.ds(HF + k * LANES, LANES)] = m * e

                    @pl.when(ch + 2 < SEC)
                    def _prefetch():
                        pltpu.async_copy(h2.at[ssec.at[ch + 2]], rows, gsem)

                    pltpu.sync_copy(em, acc.at[dsec.at[ch]], add=True)

    plsc.subcore_barrier()

    @pl.loop(0, NFIN)
    def _fin(p):
        nb = s * NPT + p * FB
        pltpu.sync_copy(acc.at[pl.ds(nb, FB)], accb)

        @plsc.parallel_loop(0, FB, unroll=8)
        def _node(i):
            for k in range(KS):
                d = accb[i, pl.ds(k * LANES, LANES)]
                u = accb[i, pl.ds(HF + k * LANES, LANES)]
                outb[i, pl.ds(k * LANES, LANES)] = u / (d + 1e-16)

        pltpu.sync_copy(outb, out.at[c, pl.ds(nb, FB)])


_sc_aggregate = pl.kernel(
    _agg_body,
    out_type=jax.ShapeDtypeStruct((NC, NP, HF), jnp.float32),
    mesh=plsc.VectorSubcoreMesh(core_axis_name="c", subcore_axis_name="s"),
    compiler_params=pltpu.CompilerParams(use_tc_tiling_on_sc=False),
    scratch_types=[
        pltpu.VMEM_SHARED((NP, 2 * HF), jnp.float32),
        pltpu.VMEM((SEC, B), jnp.int32),
        pltpu.VMEM((SEC, B), jnp.int32),
        pltpu.VMEM((SEC, B), jnp.int32),
        pltpu.VMEM((SEC, B), jnp.int32),
        pltpu.VMEM((B, HF), jnp.float32),
        pltpu.VMEM((B, HF), jnp.float32),
        pltpu.VMEM((B, 2 * HF), jnp.float32),
        pltpu.VMEM((FB, 2 * HF), jnp.float32),
        pltpu.VMEM((FB, HF), jnp.float32),
        pltpu.SemaphoreType.DMA,
        pltpu.SemaphoreType.DMA,
        pltpu.SemaphoreType.DMA,
        pltpu.SemaphoreType.DMA,
    ],
)


def _mlp_body(final, aggr_ref, h_ref, wa_ref, ba_ref, gs_ref, be_ref,
              wb_ref, bb_ref, o_ref):
    a = jnp.concatenate([aggr_ref[0], aggr_ref[1]], axis=-1) + h_ref[...]
    t = jnp.dot(a, wa_ref[...], preferred_element_type=jnp.float32)
    t = t + ba_ref[...]
    t = gs_ref[...] * (t * (1.0 / jnp.sqrt(1.0 + BN_EPS))) + be_ref[...]
    t = jnp.maximum(t, 0.0)
    o = jnp.dot(t, wb_ref[...], preferred_element_type=jnp.float32)
    o = o + bb_ref[...]
    if final == "relu":
        o_ref[...] = jnp.maximum(o, 0.0)
    else:
        m = jnp.max(o, axis=1, keepdims=True)
        ex = jnp.exp(o - m)
        lse = jnp.log(jnp.sum(ex, axis=1, keepdims=True)) + m
        o_ref[...] = o - lse


def _mlp(aggr, h, wa, ba, g, be, wb, bb, final):
    bn = 1000
    fmid = wa.shape[1]
    fout = wb.shape[1]
    grid = (N // bn,)
    return pl.pallas_call(
        functools.partial(_mlp_body, final),
        grid=grid,
        in_specs=[
            pl.BlockSpec((NC, bn, HF), lambda i: (0, i, 0)),
            pl.BlockSpec((bn, F), lambda i: (i, 0)),
            pl.BlockSpec((F, fmid), lambda i: (0, 0)),
            pl.BlockSpec((1, fmid), lambda i: (0, 0)),
            pl.BlockSpec((1, fmid), lambda i: (0, 0)),
            pl.BlockSpec((1, fmid), lambda i: (0, 0)),
            pl.BlockSpec((fmid, fout), lambda i: (0, 0)),
            pl.BlockSpec((1, fout), lambda i: (0, 0)),
        ],
        out_specs=pl.BlockSpec((bn, fout), lambda i: (i, 0)),
        out_shape=jax.ShapeDtypeStruct((N, fout), jnp.float32),
    )(aggr, h, wa, ba.reshape(1, -1), g.reshape(1, -1), be.reshape(1, -1),
      wb, bb.reshape(1, -1))


def kernel(x, edge_index, W1a, b1a, g1, be1, W1b, b1b,
           W2a, b2a, g2, be2, W2b, b2b):
    srcp = jnp.pad(edge_index[0], (0, EPAD - E))
    dstp = jnp.pad(edge_index[1], (0, EPAD - E), constant_values=N)
    sd = (jnp.stack([srcp, dstp])
          .reshape(2, NS, NSEC, SEC, B)
          .transpose(1, 2, 0, 3, 4))
    aggr1 = _sc_aggregate(x.reshape(2 * N, HF), sd)[:, :N]
    h1 = _mlp(aggr1, x, W1a, b1a, g1, be1, W1b, b1b, final="relu")
    aggr2 = _sc_aggregate(h1.reshape(2 * N, HF), sd)[:, :N]
    return _mlp(aggr2, h1, W2a, b2a, g2, be2, W2b, b2b, final="logsoftmax")


# speedup vs baseline: 2.5788x; 1.0438x over previous
import functools

import jax
import jax.numpy as jnp
from jax import lax
from jax.experimental import pallas as pl
from jax.experimental.pallas import tpu as pltpu
from jax.experimental.pallas import tpu_sc as plsc

N = 10000
E = 320000
F = 128
HF = 64
EPS = 1e-7
BN_EPS = 1e-5

NC = 2
NS = 16
LANES = 16
KS = HF // LANES

B = 64
EPT = 20480
EPAD = EPT * NS
NCHUNK = EPT // B
SEC = 16
NSEC = NCHUNK // SEC
NP = 10240
NPT = NP // NS
FB = 64
NFIN = NPT // FB


def _agg_body(h2, sd_hbm, out,
              acc,
              ssec0, ssec1, dsec0, dsec1, rows0, rows1, em0, em1,
              accb, outb, ssem0, ssem1, gsem0, gsem1, zsem0, zsem1):
    c = lax.axis_index("c")
    s = lax.axis_index("s")
    sbufs = ((ssec0, dsec0, ssem0), (ssec1, dsec1, ssem1))
    gbufs = ((rows0, gsem0), (rows1, gsem1))
    zbufs = ((em0, zsem0), (em1, zsem1))

    @plsc.parallel_loop(0, FB, unroll=8)
    def _zero(i):
        for k in range(2 * KS):
            accb[i, pl.ds(k * LANES, LANES)] = jnp.zeros((LANES,), jnp.float32)

    @pl.loop(0, NFIN)
    def _zcopy(p):
        pltpu.sync_copy(accb, acc.at[pl.ds(s * NPT + p * FB, FB)])

    plsc.subcore_barrier()

    pltpu.async_copy(sd_hbm.at[s, 0, 0], ssec0, ssem0)
    pltpu.async_copy(sd_hbm.at[s, 0, 1], dsec0, ssem0)

    @pl.loop(0, NSEC, step=2)
    def _sec(t):
        for sb in range(2):
            sec = t + sb
            ssec, dsec, ssem = sbufs[sb]
            nssec, ndsec, nssem = sbufs[1 - sb]

            pltpu.make_async_copy(sd_hbm.at[s, sec, 0], ssec, ssem).wait()
            pltpu.make_async_copy(sd_hbm.at[s, sec, 1], dsec, ssem).wait()

            @pl.when(sec + 1 < NSEC)
            def _next_sec():
                pltpu.async_copy(sd_hbm.at[s, sec + 1, 0], nssec, nssem)
                pltpu.async_copy(sd_hbm.at[s, sec + 1, 1], ndsec, nssem)

            @plsc.parallel_loop(0, SEC, unroll=4)
            def _gidx(g):
                for k in range(B // LANES):
                    sv = ssec[g, pl.ds(k * LANES, LANES)]
                    ssec[g, pl.ds(k * LANES, LANES)] = sv * 2 + c

            pltpu.async_copy(h2.at[ssec.at[0]], rows0, gsem0)
            pltpu.async_copy(h2.at[ssec.at[1]], rows1, gsem1)

            @pl.loop(0, SEC, step=2)
            def _chunk(q):
                for rb in range(2):
                    ch = q + rb
                    rows, gsem = gbufs[rb]
                    em, zsem = zbufs[rb]

                    pltpu.make_async_copy(h2.at[ssec.at[ch]], rows, gsem).wait()

                    @pl.when(ch >= 2)
                    def _drain():
                        pltpu.make_async_copy(
                            em, acc.at[dsec.at[ch]], zsem).wait()

                    @plsc.parallel_loop(0, B, unroll=8)
                    def _edge(i):
                        for k in range(KS):
                            v = rows[i, pl.ds(k * LANES, LANES)]
                            m = jnp.maximum(v, 0.0) + EPS
                            e = jnp.exp(m)
                            em[i, pl.ds(k * LANES, LANES)] = e
                            em[i, pl.ds(HF + k * LANES, LANES)] = m * e

                    @pl.when(ch + 2 < SEC)
                    def _prefetch():
                        pltpu.async_copy(h2.at[ssec.at[ch + 2]], rows, gsem)

                    pltpu.async_copy(em, acc.at[dsec.at[ch]], zsem, add=True)

            for em, zsem in zbufs:
                pltpu.make_async_copy(em, acc.at[dsec.at[0]], zsem).wait()

    plsc.subcore_barrier()

    @pl.loop(0, NFIN)
    def _fin(p):
        nb = s * NPT + p * FB
        pltpu.sync_copy(acc.at[pl.ds(nb, FB)], accb)

        @plsc.parallel_loop(0, FB, unroll=8)
        def _node(i):
            for k in range(KS):
                d = accb[i, pl.ds(k * LANES, LANES)]
                u = accb[i, pl.ds(HF + k * LANES, LANES)]
                outb[i, pl.ds(k * LANES, LANES)] = u / (d + 1e-16)

        pltpu.sync_copy(outb, out.at[c, pl.ds(nb, FB)])


_sc_aggregate = pl.kernel(
    _agg_body,
    out_type=jax.ShapeDtypeStruct((NC, NP, HF), jnp.float32),
    mesh=plsc.VectorSubcoreMesh(core_axis_name="c", subcore_axis_name="s"),
    compiler_params=pltpu.CompilerParams(use_tc_tiling_on_sc=False),
    scratch_types=[
        pltpu.VMEM_SHARED((NP, 2 * HF), jnp.float32),
        pltpu.VMEM((SEC, B), jnp.int32),
        pltpu.VMEM((SEC, B), jnp.int32),
        pltpu.VMEM((SEC, B), jnp.int32),
        pltpu.VMEM((SEC, B), jnp.int32),
        pltpu.VMEM((B, HF), jnp.float32),
        pltpu.VMEM((B, HF), jnp.float32),
        pltpu.VMEM((B, 2 * HF), jnp.float32),
        pltpu.VMEM((B, 2 * HF), jnp.float32),
        pltpu.VMEM((FB, 2 * HF), jnp.float32),
        pltpu.VMEM((FB, HF), jnp.float32),
        pltpu.SemaphoreType.DMA,
        pltpu.SemaphoreType.DMA,
        pltpu.SemaphoreType.DMA,
        pltpu.SemaphoreType.DMA,
        pltpu.SemaphoreType.DMA,
        pltpu.SemaphoreType.DMA,
    ],
)


def _mlp_body(final, aggr_ref, h_ref, wa_ref, ba_ref, gs_ref, be_ref,
              wb_ref, bb_ref, o_ref):
    a = jnp.concatenate([aggr_ref[0], aggr_ref[1]], axis=-1) + h_ref[...]
    t = jnp.dot(a, wa_ref[...], preferred_element_type=jnp.float32)
    t = t + ba_ref[...]
    t = gs_ref[...] * (t * (1.0 / jnp.sqrt(1.0 + BN_EPS))) + be_ref[...]
    t = jnp.maximum(t, 0.0)
    o = jnp.dot(t, wb_ref[...], preferred_element_type=jnp.float32)
    o = o + bb_ref[...]
    if final == "relu":
        o_ref[...] = jnp.maximum(o, 0.0)
    else:
        m = jnp.max(o, axis=1, keepdims=True)
        ex = jnp.exp(o - m)
        lse = jnp.log(jnp.sum(ex, axis=1, keepdims=True)) + m
        o_ref[...] = o - lse


def _mlp(aggr, h, wa, ba, g, be, wb, bb, final):
    bn = 1000
    fmid = wa.shape[1]
    fout = wb.shape[1]
    grid = (N // bn,)
    return pl.pallas_call(
        functools.partial(_mlp_body, final),
        grid=grid,
        in_specs=[
            pl.BlockSpec((NC, bn, HF), lambda i: (0, i, 0)),
            pl.BlockSpec((bn, F), lambda i: (i, 0)),
            pl.BlockSpec((F, fmid), lambda i: (0, 0)),
            pl.BlockSpec((1, fmid), lambda i: (0, 0)),
            pl.BlockSpec((1, fmid), lambda i: (0, 0)),
            pl.BlockSpec((1, fmid), lambda i: (0, 0)),
            pl.BlockSpec((fmid, fout), lambda i: (0, 0)),
            pl.BlockSpec((1, fout), lambda i: (0, 0)),
        ],
        out_specs=pl.BlockSpec((bn, fout), lambda i: (i, 0)),
        out_shape=jax.ShapeDtypeStruct((N, fout), jnp.float32),
    )(aggr, h, wa, ba.reshape(1, -1), g.reshape(1, -1), be.reshape(1, -1),
      wb, bb.reshape(1, -1))


def kernel(x, edge_index, W1a, b1a, g1, be1, W1b, b1b,
           W2a, b2a, g2, be2, W2b, b2b):
    srcp = jnp.pad(edge_index[0], (0, EPAD - E))
    dstp = jnp.pad(edge_index[1], (0, EPAD - E), constant_values=N)
    sd = (jnp.stack([srcp, dstp])
          .reshape(2, NS, NSEC, SEC, B)
          .transpose(1, 2, 0, 3, 4))
    aggr1 = _sc_aggregate(x.reshape(2 * N, HF), sd)[:, :N]
    h1 = _mlp(aggr1, x, W1a, b1a, g1, be1, W1b, b1b, final="relu")
    aggr2 = _sc_aggregate(h1.reshape(2 * N, HF), sd)[:, :N]
    return _mlp(aggr2, h1, W2a, b2a, g2, be2, W2b, b2b, final="logsoftmax")
